# baseline (speedup 1.0000x reference)
import functools

import jax
import jax.numpy as jnp
from jax import lax
from jax.experimental import pallas as pl
from jax.experimental.pallas import tpu as pltpu
from jax.experimental.pallas import tpu_sc as plsc

N_ROWS = 16384
N_COLS = 256
NC = 2
NS = 16
L = 16
NW = NC * NS
ROWS_PER_W = N_ROWS // NW
G = 64
N_GROUPS = ROWS_PER_W // G


def _sc_excl_cumsum(x_flat):
    mesh = plsc.VectorSubcoreMesh(core_axis_name="c", subcore_axis_name="s")

    @functools.partial(
        pl.kernel,
        mesh=mesh,
        out_type=jax.ShapeDtypeStruct((N_ROWS * N_COLS,), jnp.float32),
        scratch_types=[
            pltpu.VMEM((G * N_COLS,), jnp.float32),
            pltpu.VMEM((G * N_COLS,), jnp.float32),
        ],
        compiler_params=pltpu.CompilerParams(needs_layout_passes=False),
    )
    def k(x_hbm, out_hbm, ibuf, obuf):
        wid = lax.axis_index("s") * NC + lax.axis_index("c")
        base = wid * (ROWS_PER_W * N_COLS)
        row_base = lax.iota(jnp.int32, L) * N_COLS

        sg_bases = [row_base + sg * (L * N_COLS) for sg in range(G // L)]

        def group(g, carry):
            goff = base + g * (G * N_COLS)
            pltpu.sync_copy(x_hbm.at[pl.ds(goff, G * N_COLS)], ibuf)

            def col(c, accs):
                out = []
                for sg_base, acc in zip(sg_bases, accs):
                    idx = sg_base + c
                    v = plsc.load_gather(ibuf, [idx])
                    plsc.store_scatter(obuf, [idx], acc)
                    out.append(acc + v)
                return tuple(out)

            zero = jnp.zeros((L,), jnp.float32)
            lax.fori_loop(0, N_COLS, col, tuple(zero for _ in sg_bases),
                          unroll=8)
            pltpu.sync_copy(obuf, out_hbm.at[pl.ds(goff, G * N_COLS)])
            return carry

        lax.fori_loop(0, N_GROUPS, group, 0)

    return k(x_flat)


def kernel(x):
    out_flat = _sc_excl_cumsum(x.reshape(-1))
    return out_flat.reshape(N_ROWS, N_COLS)


# speedup vs baseline: 2.4067x; 2.4067x over previous
import functools

import jax
import jax.numpy as jnp
from jax import lax
from jax.experimental import pallas as pl
from jax.experimental.pallas import tpu as pltpu
from jax.experimental.pallas import tpu_sc as plsc

N_ROWS = 16384
N_COLS = 256
NC = 2
NS = 16
L = 16
NW = NC * NS
ROWS_PER_W = N_ROWS // NW
G = 64
N_GROUPS = ROWS_PER_W // G


def _sc_excl_cumsum(x_flat):
    mesh = plsc.VectorSubcoreMesh(core_axis_name="c", subcore_axis_name="s")

    @functools.partial(
        pl.kernel,
        mesh=mesh,
        out_type=jax.ShapeDtypeStruct((N_ROWS * N_COLS,), jnp.float32),
        scratch_types=[
            pltpu.VMEM((G * N_COLS,), jnp.float32),
            pltpu.VMEM((G * N_COLS,), jnp.float32),
        ],
        compiler_params=pltpu.CompilerParams(needs_layout_passes=False),
    )
    def k(x_hbm, out_hbm, ibuf, obuf):
        wid = lax.axis_index("s") * NC + lax.axis_index("c")
        base = wid * (ROWS_PER_W * N_COLS)
        row_base = lax.iota(jnp.int32, L) * N_COLS

        riota = lax.iota(jnp.int32, L)
        skew = riota * (N_COLS - 1)
        sg_bases = [skew + sg * (L * N_COLS) for sg in range(G // L)]

        def masked_step(t, accs):
            m = (riota <= t) & (t < riota + N_COLS)
            out = []
            for sg_base, acc in zip(sg_bases, accs):
                idx = sg_base + t
                v = plsc.load_gather(ibuf, [idx], mask=m)
                plsc.store_scatter(obuf, [idx], acc, mask=m)
                out.append(acc + jnp.where(m, v, 0.0))
            return tuple(out)

        def step(t, accs):
            out = []
            for sg_base, acc in zip(sg_bases, accs):
                idx = sg_base + t
                v = plsc.load_gather(ibuf, [idx])
                plsc.store_scatter(obuf, [idx], acc)
                out.append(acc + v)
            return tuple(out)

        def group(g, carry):
            goff = base + g * (G * N_COLS)
            pltpu.sync_copy(x_hbm.at[pl.ds(goff, G * N_COLS)], ibuf)
            zero = jnp.zeros((L,), jnp.float32)
            accs = tuple(zero for _ in sg_bases)
            accs = lax.fori_loop(0, L, masked_step, accs)
            accs = lax.fori_loop(L, N_COLS, step, accs, unroll=8)
            lax.fori_loop(N_COLS, N_COLS + L - 1, masked_step, accs)
            pltpu.sync_copy(obuf, out_hbm.at[pl.ds(goff, G * N_COLS)])
            return carry

        lax.fori_loop(0, N_GROUPS, group, 0)

    return k(x_flat)


def kernel(x):
    out_flat = _sc_excl_cumsum(x.reshape(-1))
    return out_flat.reshape(N_ROWS, N_COLS)


# speedup vs baseline: 4.2296x; 1.7574x over previous
import functools

import jax
import jax.numpy as jnp
from jax import lax
from jax.experimental import pallas as pl
from jax.experimental.pallas import tpu as pltpu
from jax.experimental.pallas import tpu_sc as plsc

N_ROWS = 16384
N_COLS = 256
NC = 2
NS = 16
L = 16
NW = NC * NS
ROWS_PER_W = N_ROWS // NW
G = 64
N_GROUPS = ROWS_PER_W // G


def _sc_excl_cumsum(x):
    mesh = plsc.VectorSubcoreMesh(core_axis_name="c", subcore_axis_name="s")

    @functools.partial(
        pl.kernel,
        mesh=mesh,
        out_type=jax.ShapeDtypeStruct((N_ROWS, N_COLS), jnp.float32),
        scratch_types=[
            pltpu.VMEM((G, N_COLS), jnp.float32),
            pltpu.VMEM((G, N_COLS), jnp.float32),
        ],
        compiler_params=pltpu.CompilerParams(needs_layout_passes=False),
    )
    def k(x_hbm, out_hbm, ibuf, obuf):
        wid = lax.axis_index("s") * NC + lax.axis_index("c")
        row0 = wid * ROWS_PER_W
        riota = lax.iota(jnp.int32, L)
        sg_rows = [riota + sg * L for sg in range(G // L)]

        def masked_step(t, accs):
            m = (riota <= t) & (t < riota + N_COLS)
            col = t - riota
            out = []
            for rows, acc in zip(sg_rows, accs):
                v = plsc.load_gather(ibuf, [rows, col], mask=m)
                plsc.store_scatter(obuf, [rows, col], acc, mask=m)
                out.append(acc + jnp.where(m, v, 0.0))
            return tuple(out)

        def step(t, accs):
            col = t - riota
            out = []
            for rows, acc in zip(sg_rows, accs):
                v = plsc.load_gather(ibuf, [rows, col])
                plsc.store_scatter(obuf, [rows, col], acc)
                out.append(acc + v)
            return tuple(out)

        def group(g, carry):
            r0 = row0 + g * G
            pltpu.sync_copy(x_hbm.at[pl.ds(r0, G), :], ibuf)
            zero = jnp.zeros((L,), jnp.float32)
            accs = tuple(zero for _ in sg_rows)
            accs = lax.fori_loop(0, L, masked_step, accs)
            accs = lax.fori_loop(L, N_COLS, step, accs, unroll=8)
            lax.fori_loop(N_COLS, N_COLS + L - 1, masked_step, accs)
            pltpu.sync_copy(obuf, out_hbm.at[pl.ds(r0, G), :])
            return carry

        lax.fori_loop(0, N_GROUPS, group, 0)

    return k(x)


def kernel(x):
    return _sc_excl_cumsum(x)


# speedup vs baseline: 5.0866x; 1.2026x over previous
import functools

import jax
import jax.numpy as jnp
from jax import lax
from jax.experimental import pallas as pl
from jax.experimental.pallas import tpu as pltpu
from jax.experimental.pallas import tpu_sc as plsc

N_ROWS = 16384
N_COLS = 256
NC = 2
NS = 16
L = 16
NW = NC * NS
ROWS_PER_W = N_ROWS // NW
G = 64
N_GROUPS = ROWS_PER_W // G


def _sc_excl_cumsum(x):
    mesh = plsc.VectorSubcoreMesh(core_axis_name="c", subcore_axis_name="s")

    @functools.partial(
        pl.kernel,
        mesh=mesh,
        out_type=jax.ShapeDtypeStruct((N_ROWS, N_COLS), jnp.float32),
        scratch_types=[
            pltpu.VMEM((G, N_COLS), jnp.float32),
            pltpu.VMEM((G, N_COLS), jnp.float32),
            pltpu.VMEM((G, N_COLS), jnp.float32),
            pltpu.VMEM((G, N_COLS), jnp.float32),
            pltpu.SemaphoreType.DMA,
            pltpu.SemaphoreType.DMA,
            pltpu.SemaphoreType.DMA,
            pltpu.SemaphoreType.DMA,
        ],
        compiler_params=pltpu.CompilerParams(needs_layout_passes=False),
    )
    def k(x_hbm, out_hbm, ib0, ib1, ob0, ob1, si0, si1, so0, so1):
        ibufs, obufs = (ib0, ib1), (ob0, ob1)
        sins, souts = (si0, si1), (so0, so1)
        wid = lax.axis_index("s") * NC + lax.axis_index("c")
        row0 = wid * ROWS_PER_W
        riota = lax.iota(jnp.int32, L)
        sg_rows = [riota + sg * L for sg in range(G // L)]

        def in_copy(g):
            r0 = row0 + g * G
            return pltpu.make_async_copy(
                x_hbm.at[pl.ds(r0, G), :], ibufs[g % 2], sins[g % 2])

        def out_copy(g):
            r0 = row0 + g * G
            return pltpu.make_async_copy(
                obufs[g % 2], out_hbm.at[pl.ds(r0, G), :], souts[g % 2])

        def compute(ibuf, obuf):
            def masked_step(t, accs):
                m = (riota <= t) & (t < riota + N_COLS)
                col = t - riota
                out = []
                for rows, acc in zip(sg_rows, accs):
                    v = plsc.load_gather(ibuf, [rows, col], mask=m)
                    plsc.store_scatter(obuf, [rows, col], acc, mask=m)
                    out.append(acc + jnp.where(m, v, 0.0))
                return tuple(out)

            def step(t, accs):
                col = t - riota
                out = []
                for rows, acc in zip(sg_rows, accs):
                    v = plsc.load_gather(ibuf, [rows, col])
                    plsc.store_scatter(obuf, [rows, col], acc)
                    out.append(acc + v)
                return tuple(out)

            zero = jnp.zeros((L,), jnp.float32)
            accs = tuple(zero for _ in sg_rows)
            accs = lax.fori_loop(0, L, masked_step, accs)
            accs = lax.fori_loop(L, N_COLS, step, accs, unroll=8)
            lax.fori_loop(N_COLS, N_COLS + L - 1, masked_step, accs)

        in_copy(0).start()
        for g in range(N_GROUPS):
            if g + 1 < N_GROUPS:
                in_copy(g + 1).start()
            in_copy(g).wait()
            if g >= 2:
                out_copy(g - 2).wait()
            compute(ibufs[g % 2], obufs[g % 2])
            out_copy(g).start()
        out_copy(N_GROUPS - 2).wait()
        out_copy(N_GROUPS - 1).wait()

    return k(x)


def kernel(x):
    return _sc_excl_cumsum(x)


# speedup vs baseline: 5.2284x; 1.0279x over previous
import functools

import jax
import jax.numpy as jnp
from jax import lax
from jax.experimental import pallas as pl
from jax.experimental.pallas import tpu as pltpu
from jax.experimental.pallas import tpu_sc as plsc

N_ROWS = 16384
N_COLS = 256
NC = 2
NS = 16
L = 16
NW = NC * NS
SC_ROWS = 6144
TC_ROWS = N_ROWS - SC_ROWS
ROWS_PER_W = SC_ROWS // NW
G = 64
N_GROUPS = ROWS_PER_W // G
TC_BR = 1024


def _sc_excl_cumsum(x):
    mesh = plsc.VectorSubcoreMesh(core_axis_name="c", subcore_axis_name="s")

    @functools.partial(
        pl.kernel,
        mesh=mesh,
        out_type=jax.ShapeDtypeStruct((N_ROWS, N_COLS), jnp.float32),
        scratch_types=[
            pltpu.VMEM((G, N_COLS), jnp.float32),
            pltpu.VMEM((G, N_COLS), jnp.float32),
            pltpu.VMEM((G, N_COLS), jnp.float32),
            pltpu.VMEM((G, N_COLS), jnp.float32),
            pltpu.SemaphoreType.DMA,
            pltpu.SemaphoreType.DMA,
            pltpu.SemaphoreType.DMA,
            pltpu.SemaphoreType.DMA,
        ],
        compiler_params=pltpu.CompilerParams(needs_layout_passes=False),
    )
    def k(x_hbm, out_hbm, ib0, ib1, ob0, ob1, si0, si1, so0, so1):
        ibufs, obufs = (ib0, ib1), (ob0, ob1)
        sins, souts = (si0, si1), (so0, so1)
        wid = lax.axis_index("s") * NC + lax.axis_index("c")
        row0 = wid * ROWS_PER_W
        riota = lax.iota(jnp.int32, L)
        sg_rows = [riota + sg * L for sg in range(G // L)]

        def in_copy(g):
            r0 = row0 + g * G
            return pltpu.make_async_copy(
                x_hbm.at[pl.ds(r0, G), :], ibufs[g % 2], sins[g % 2])

        def out_copy(g):
            r0 = row0 + g * G
            return pltpu.make_async_copy(
                obufs[g % 2], out_hbm.at[pl.ds(r0, G), :], souts[g % 2])

        def compute(ibuf, obuf):
            def masked_step(t, accs):
                m = (riota <= t) & (t < riota + N_COLS)
                col = t - riota
                out = []
                for rows, acc in zip(sg_rows, accs):
                    v = plsc.load_gather(ibuf, [rows, col], mask=m)
                    plsc.store_scatter(obuf, [rows, col], acc, mask=m)
                    out.append(acc + jnp.where(m, v, 0.0))
                return tuple(out)

            def step(t, accs):
                col = t - riota
                out = []
                for rows, acc in zip(sg_rows, accs):
                    v = plsc.load_gather(ibuf, [rows, col])
                    plsc.store_scatter(obuf, [rows, col], acc)
                    out.append(acc + v)
                return tuple(out)

            zero = jnp.zeros((L,), jnp.float32)
            accs = tuple(zero for _ in sg_rows)
            accs = lax.fori_loop(0, L, masked_step, accs)
            accs = lax.fori_loop(L, N_COLS, step, accs, unroll=8)
            lax.fori_loop(N_COLS, N_COLS + L - 1, masked_step, accs)

        in_copy(0).start()
        for g in range(N_GROUPS):
            if g + 1 < N_GROUPS:
                in_copy(g + 1).start()
            in_copy(g).wait()
            if g >= 2:
                out_copy(g - 2).wait()
            compute(ibufs[g % 2], obufs[g % 2])
            out_copy(g).start()
        out_copy(N_GROUPS - 2).wait()
        out_copy(N_GROUPS - 1).wait()

    return k(x)


def _tc_excl_cumsum_tail(x):
    def body(x_ref, o_ref):
        r = lax.broadcasted_iota(jnp.int32, (N_COLS, N_COLS), 0)
        c = lax.broadcasted_iota(jnp.int32, (N_COLS, N_COLS), 1)
        tri = (r < c).astype(jnp.float32)
        o_ref[...] = jnp.dot(x_ref[...], tri,
                             preferred_element_type=jnp.float32)

    return pl.pallas_call(
        body,
        grid=(TC_ROWS // TC_BR,),
        in_specs=[pl.BlockSpec((TC_BR, N_COLS),
                               lambda i: (SC_ROWS // TC_BR + i, 0))],
        out_specs=pl.BlockSpec((TC_BR, N_COLS), lambda i: (i, 0)),
        out_shape=jax.ShapeDtypeStruct((TC_ROWS, N_COLS), jnp.float32),
    )(x)


def kernel(x):
    sc_out = _sc_excl_cumsum(x)
    tc_out = _tc_excl_cumsum_tail(x)
    return lax.dynamic_update_slice(sc_out, tc_out, (SC_ROWS, 0))
